# baseline (speedup 1.0000x reference)
import functools

import jax
import jax.numpy as jnp
from jax import lax
from jax.experimental import pallas as pl
from jax.experimental.pallas import tpu as pltpu
from jax.experimental.pallas import tpu_sc as plsc

_NC = 2
_NS = 16
_K = 80
_GP = 64
_KD = 40
_ZR = 24
_ZC = 48


def _zero_stripe(ref, zbuf, base, RP, N, t):
    @pl.loop(0, RP // _ZR)
    def _(r):
        pltpu.sync_copy(zbuf, ref.at[pl.ds(base + r * _ZR, _ZR)])

    tail = N - _NS * RP
    if tail:
        @pl.when(t == 0)
        def _():
            pltpu.sync_copy(zbuf.at[pl.ds(0, tail)],
                            ref.at[pl.ds(_NS * RP, tail)])


def _sc_agg_body(G, RP, N, F, z_hbm, srccat, dstr, out_hbm,
                 acc, srcb, dstb, rows0, rows1, sem0, sem1, sem2):
    c = lax.axis_index("c")
    t = lax.axis_index("s")
    base = t * RP
    tail = N - _NS * RP

    zero16 = jnp.zeros((16,), jnp.float32)

    @pl.loop(0, _ZC)
    def _(r):
        for c16 in range(F // 16):
            rows1[r, pl.ds(c16 * 16, 16)] = zero16

    @pl.loop(0, RP // _ZC)
    def _(r):
        pltpu.sync_copy(rows1.at[pl.ds(0, _ZC)],
                        acc.at[pl.ds(base + r * _ZC, _ZC)])

    if tail:
        @pl.when(t == 0)
        def _():
            pltpu.sync_copy(rows1.at[pl.ds(0, tail)],
                            acc.at[pl.ds(_NS * RP, tail)])

    plsc.subcore_barrier()

    for p in range(G // _GP):
        pltpu.sync_copy(srccat.at[c, t, pl.ds(p * _GP, _GP)], srcb)
        pltpu.sync_copy(dstr.at[t, pl.ds(p * _GP, _GP)], dstb)

        @pl.loop(0, _GP)
        def _(g):
            pltpu.async_copy(z_hbm.at[srcb.at[g]], rows0, sem0).wait()
            pltpu.sync_copy(rows0, acc.at[dstb.at[g]], add=True)

    plsc.subcore_barrier()

    pltpu.sync_copy(acc.at[pl.ds(base, RP)],
                    out_hbm.at[pl.ds(c * N + base, RP)])
    if tail:
        @pl.when(t == 0)
        def _():
            pltpu.sync_copy(acc.at[pl.ds(_NS * RP, tail)],
                            out_hbm.at[pl.ds(c * N + _NS * RP, tail)])


@functools.lru_cache(maxsize=None)
def _make_sc_agg(N, E, F):
    EP = E // _NS
    GC = -(-EP // _K)
    G = -(-GC // _GP) * _GP
    RP = (N // _NS) // _ZC * _ZC

    mesh = plsc.VectorSubcoreMesh(core_axis_name="c", subcore_axis_name="s",
                                  num_cores=_NC, num_subcores=_NS)
    scratch = (
        pltpu.VMEM_SHARED((N + 8, F), jnp.float32),
        pltpu.VMEM((_GP, _K), jnp.int32),
        pltpu.VMEM((_GP, _K), jnp.int32),
        pltpu.VMEM((_K, F), jnp.float32),
        pltpu.VMEM((_K, F), jnp.float32),
        pltpu.SemaphoreType.DMA,
        pltpu.SemaphoreType.DMA,
        pltpu.SemaphoreType.DMA,
    )
    body = functools.partial(_sc_agg_body, G, RP, N, F)
    return pl.kernel(body,
                     out_type=jax.ShapeDtypeStruct((2 * N, F), jnp.float32),
                     mesh=mesh, scratch_types=scratch)


def _sc_deg_body(G, RP, N, F, dstr, deg_hbm, degsh, dstb, onesb, zbuf):
    c = lax.axis_index("c")
    t = lax.axis_index("s")
    base = t * RP
    tail = N - _NS * RP

    zero16 = jnp.zeros((16,), jnp.float32)
    one16 = jnp.full((16,), 1.0, jnp.float32)

    @pl.loop(0, _ZR)
    def _(r):
        for c16 in range(F // 16):
            zbuf[r, pl.ds(c16 * 16, 16)] = zero16

    @pl.loop(0, _KD)
    def _(r):
        for c16 in range(F // 16):
            onesb[r, pl.ds(c16 * 16, 16)] = one16

    _zero_stripe(degsh, zbuf, base, RP, N, t)

    plsc.subcore_barrier()

    pltpu.sync_copy(dstr.at[c, t], dstb)

    @pl.loop(0, G)
    def _(g):
        pltpu.sync_copy(onesb, degsh.at[dstb.at[g]], add=True)

    plsc.subcore_barrier()

    pltpu.sync_copy(degsh.at[pl.ds(base, RP)],
                    deg_hbm.at[pl.ds(c * N + base, RP)])
    if tail:
        @pl.when(t == 0)
        def _():
            pltpu.sync_copy(degsh.at[pl.ds(_NS * RP, tail)],
                            deg_hbm.at[pl.ds(c * N + _NS * RP, tail)])


@functools.lru_cache(maxsize=None)
def _make_sc_deg(N, E, F):
    EP = E // (_NC * _NS)
    G = EP // _KD
    RP = (N // _NS) // _ZR * _ZR

    mesh = plsc.VectorSubcoreMesh(core_axis_name="c", subcore_axis_name="s",
                                  num_cores=_NC, num_subcores=_NS)
    scratch = (
        pltpu.VMEM_SHARED((N, F), jnp.float32),
        pltpu.VMEM((G, _KD), jnp.int32),
        pltpu.VMEM((_KD, F), jnp.float32),
        pltpu.VMEM((_ZR, F), jnp.float32),
    )
    body = functools.partial(_sc_deg_body, G, RP, N, F)
    return pl.kernel(
        body,
        out_type=jax.ShapeDtypeStruct((_NC * N, F), jnp.float32),
        mesh=mesh, scratch_types=scratch)


def _l1_body(x_ref, ws_ref, bs_ref, wn_ref, s_ref, z_ref):
    j = pl.program_id(1)

    @pl.when(j == 0)
    def _():
        s_ref[...] = jnp.maximum(x_ref[...] @ ws_ref[...] + bs_ref[...], 0.0)

    z_ref[...] = x_ref[...] @ wn_ref[...]


def _l2_body(s1_ref, alo_ref, ahi_ref, dega_ref, degb_ref, b1n_ref, ws_ref,
             bs_ref, wn_ref, s2_ref, z2_ref, n1_ref):
    j = pl.program_id(1)

    @pl.when(j == 0)
    def _():
        d = jnp.maximum(dega_ref[:, 0:1] + degb_ref[:, 0:1], 1.0)
        a = jnp.concatenate([alo_ref[...], ahi_ref[...]], axis=1) / d
        n1 = jnp.maximum(a + b1n_ref[...], 0.0)
        n1_ref[...] = n1
        ws = ws_ref[...]
        s2_ref[...] = jnp.maximum(
            s1_ref[...] @ ws[:256] + n1 @ ws[256:] + bs_ref[...], 0.0)

    wn = wn_ref[...]
    z2_ref[...] = s1_ref[...] @ wn[:256] + n1_ref[...] @ wn[256:]


def _l3_body(s2_ref, alo_ref, ahi_ref, dega_ref, degb_ref, b2n_ref, wc_ref,
             bc_ref, o_ref):
    d = jnp.maximum(dega_ref[:, 0:1] + degb_ref[:, 0:1], 1.0)
    a = jnp.concatenate([alo_ref[...], ahi_ref[...]], axis=1) / d
    n2 = jnp.maximum(a + b2n_ref[...], 0.0)
    s2 = s2_ref[...]
    ss = (jnp.sum(s2 * s2, axis=1, keepdims=True)
          + jnp.sum(n2 * n2, axis=1, keepdims=True))
    r = 1.0 / jnp.maximum(jnp.sqrt(ss), 1e-12)
    wc = wc_ref[...]
    o_ref[...] = (s2 * r) @ wc[:256] + (n2 * r) @ wc[256:] + bc_ref[...]


def kernel(x, edge_index, W1_self, b1_self, W1_neigh, b1_neigh,
           W2_self, b2_self, W2_neigh, b2_neigh, W_cls, b_cls):
    N, Din = x.shape
    E = edge_index.shape[1]
    H = W1_self.shape[1]
    Dout = W_cls.shape[1]
    F = H // 2
    NB = 10
    BM = N // NB
    EP = E // _NS
    GC = -(-EP // _K)
    G = -(-GC // _GP) * _GP
    GD = (E // (_NC * _NS)) // _KD

    src = edge_index[0]
    dst = edge_index[1]
    pad = ((0, 0), (0, 0), (0, G * _K - EP))
    srccat = jnp.pad(
        jnp.concatenate([src, src + N]).reshape(_NC, _NS, EP),
        pad).reshape(_NC, _NS, G, _K)
    dstr = jnp.pad(dst.reshape(1, _NS, EP), pad,
                   constant_values=N)[0].reshape(_NS, G, _K)
    dstr32 = dst.reshape(_NC, _NS, GD, _KD)

    full = lambda shape: pl.BlockSpec(shape, lambda i, j: (0, 0))

    deg2 = _make_sc_deg(N, E, F)(dstr32)

    s1, z1 = pl.pallas_call(
        _l1_body,
        grid=(NB, _NC),
        in_specs=[
            pl.BlockSpec((BM, Din), lambda i, j: (i, 0)),
            full((Din, H)),
            full((1, H)),
            pl.BlockSpec((Din, F), lambda i, j: (0, j)),
        ],
        out_specs=[
            pl.BlockSpec((BM, H), lambda i, j: (i, 0)),
            pl.BlockSpec((BM, F), lambda i, j: (j * NB + i, 0)),
        ],
        out_shape=[
            jax.ShapeDtypeStruct((N, H), jnp.float32),
            jax.ShapeDtypeStruct((2 * N, F), jnp.float32),
        ],
    )(x, W1_self, b1_self.reshape(1, H), W1_neigh)

    agg1 = _make_sc_agg(N, E, F)(z1, srccat, dstr)

    s2, z2 = pl.pallas_call(
        _l2_body,
        grid=(NB, _NC),
        in_specs=[
            pl.BlockSpec((BM, H), lambda i, j: (i, 0)),
            pl.BlockSpec((BM, F), lambda i, j: (i, 0)),
            pl.BlockSpec((BM, F), lambda i, j: (NB + i, 0)),
            pl.BlockSpec((BM, F), lambda i, j: (i, 0)),
            pl.BlockSpec((BM, F), lambda i, j: (NB + i, 0)),
            full((1, H)),
            full((2 * H, H)),
            full((1, H)),
            pl.BlockSpec((2 * H, F), lambda i, j: (0, j)),
        ],
        out_specs=[
            pl.BlockSpec((BM, H), lambda i, j: (i, 0)),
            pl.BlockSpec((BM, F), lambda i, j: (j * NB + i, 0)),
        ],
        out_shape=[
            jax.ShapeDtypeStruct((N, H), jnp.float32),
            jax.ShapeDtypeStruct((2 * N, F), jnp.float32),
        ],
        scratch_shapes=[pltpu.VMEM((BM, H), jnp.float32)],
    )(s1, agg1, agg1, deg2, deg2, b1_neigh.reshape(1, H), W2_self,
      b2_self.reshape(1, H), W2_neigh)

    agg2 = _make_sc_agg(N, E, F)(z2, srccat, dstr)

    out = pl.pallas_call(
        _l3_body,
        grid=(NB,),
        in_specs=[
            pl.BlockSpec((BM, H), lambda i: (i, 0)),
            pl.BlockSpec((BM, F), lambda i: (i, 0)),
            pl.BlockSpec((BM, F), lambda i: (NB + i, 0)),
            pl.BlockSpec((BM, F), lambda i: (i, 0)),
            pl.BlockSpec((BM, F), lambda i: (NB + i, 0)),
            pl.BlockSpec((1, H), lambda i: (0, 0)),
            pl.BlockSpec((2 * H, Dout), lambda i: (0, 0)),
            pl.BlockSpec((1, Dout), lambda i: (0, 0)),
        ],
        out_specs=pl.BlockSpec((BM, Dout), lambda i: (i, 0)),
        out_shape=jax.ShapeDtypeStruct((N, Dout), jnp.float32),
    )(s2, agg2, agg2, deg2, deg2, b2_neigh.reshape(1, H), W_cls,
      b_cls.reshape(1, Dout))

    return out


# speedup vs baseline: 1.9822x; 1.9822x over previous
import functools

import jax
import jax.numpy as jnp
from jax import lax
from jax.experimental import pallas as pl
from jax.experimental.pallas import tpu as pltpu
from jax.experimental.pallas import tpu_sc as plsc

_NC = 2
_NS = 16
_K = 80
_KD = 40
_ZR = 24
_ZC = 48


def _zero_stripe(ref, zbuf, base, RP, N, t):
    @pl.loop(0, RP // _ZR)
    def _(r):
        pltpu.sync_copy(zbuf, ref.at[pl.ds(base + r * _ZR, _ZR)])

    tail = N - _NS * RP
    if tail:
        @pl.when(t == 0)
        def _():
            pltpu.sync_copy(zbuf.at[pl.ds(0, tail)],
                            ref.at[pl.ds(_NS * RP, tail)])


def _sc_agg_body(G, RP, N, F, z_hbm, srccat, dstr, out_hbm,
                 acc, srcb, dstb, rows0, sem0):
    c = lax.axis_index("c")
    t = lax.axis_index("s")
    base = t * RP
    tail = N - _NS * RP

    zero16 = jnp.zeros((16,), jnp.float32)

    pltpu.sync_copy(srccat.at[c, t], srcb)
    pltpu.sync_copy(dstr.at[t], dstb)

    @pl.loop(0, _ZC)
    def _(r):
        for c16 in range(F // 16):
            rows0[r, pl.ds(c16 * 16, 16)] = zero16

    @pl.loop(0, RP // _ZC)
    def _(r):
        pltpu.sync_copy(rows0.at[pl.ds(0, _ZC)],
                        acc.at[pl.ds(base + r * _ZC, _ZC)])

    if tail:
        @pl.when(t == 0)
        def _():
            pltpu.sync_copy(rows0.at[pl.ds(0, tail)],
                            acc.at[pl.ds(_NS * RP, tail)])

    plsc.subcore_barrier()

    @pl.loop(0, G)
    def _(g):
        pltpu.async_copy(z_hbm.at[srcb.at[g]], rows0, sem0).wait()
        pltpu.sync_copy(rows0, acc.at[dstb.at[g]], add=True)

    plsc.subcore_barrier()

    pltpu.sync_copy(acc.at[pl.ds(base, RP)],
                    out_hbm.at[pl.ds(c * N + base, RP)])
    if tail:
        @pl.when(t == 0)
        def _():
            pltpu.sync_copy(acc.at[pl.ds(_NS * RP, tail)],
                            out_hbm.at[pl.ds(c * N + _NS * RP, tail)])


@functools.lru_cache(maxsize=None)
def _make_sc_agg(N, E, F):
    EP = E // _NS
    G = EP // _K
    RP = (N // _NS) // _ZC * _ZC

    mesh = plsc.VectorSubcoreMesh(core_axis_name="c", subcore_axis_name="s",
                                  num_cores=_NC, num_subcores=_NS)
    scratch = (
        pltpu.VMEM_SHARED((N, F), jnp.float32),
        pltpu.VMEM((G, _K), jnp.int32),
        pltpu.VMEM((G, _K), jnp.int32),
        pltpu.VMEM((_K, F), jnp.float32),
        pltpu.SemaphoreType.DMA,
    )
    body = functools.partial(_sc_agg_body, G, RP, N, F)
    return pl.kernel(body,
                     out_type=jax.ShapeDtypeStruct((2 * N, F), jnp.float32),
                     mesh=mesh, scratch_types=scratch)


def _sc_deg_body(G, RP, N, F, dstr, deg_hbm, degsh, dstb, onesb, zbuf):
    c = lax.axis_index("c")
    t = lax.axis_index("s")
    base = t * RP
    tail = N - _NS * RP

    zero16 = jnp.zeros((16,), jnp.float32)
    one16 = jnp.full((16,), 1.0, jnp.float32)

    @pl.loop(0, _ZR)
    def _(r):
        for c16 in range(F // 16):
            zbuf[r, pl.ds(c16 * 16, 16)] = zero16

    @pl.loop(0, _KD)
    def _(r):
        for c16 in range(F // 16):
            onesb[r, pl.ds(c16 * 16, 16)] = one16

    _zero_stripe(degsh, zbuf, base, RP, N, t)

    plsc.subcore_barrier()

    pltpu.sync_copy(dstr.at[c, t], dstb)

    @pl.loop(0, G)
    def _(g):
        pltpu.sync_copy(onesb, degsh.at[dstb.at[g]], add=True)

    plsc.subcore_barrier()

    pltpu.sync_copy(degsh.at[pl.ds(base, RP)],
                    deg_hbm.at[pl.ds(c * N + base, RP)])
    if tail:
        @pl.when(t == 0)
        def _():
            pltpu.sync_copy(degsh.at[pl.ds(_NS * RP, tail)],
                            deg_hbm.at[pl.ds(c * N + _NS * RP, tail)])


@functools.lru_cache(maxsize=None)
def _make_sc_deg(N, E, F):
    EP = E // (_NC * _NS)
    G = EP // _KD
    RP = (N // _NS) // _ZR * _ZR

    mesh = plsc.VectorSubcoreMesh(core_axis_name="c", subcore_axis_name="s",
                                  num_cores=_NC, num_subcores=_NS)
    scratch = (
        pltpu.VMEM_SHARED((N, F), jnp.float32),
        pltpu.VMEM((G, _KD), jnp.int32),
        pltpu.VMEM((_KD, F), jnp.float32),
        pltpu.VMEM((_ZR, F), jnp.float32),
    )
    body = functools.partial(_sc_deg_body, G, RP, N, F)
    return pl.kernel(
        body,
        out_type=jax.ShapeDtypeStruct((_NC * N, F), jnp.float32),
        mesh=mesh, scratch_types=scratch)


def _l1_body(x_ref, ws_ref, bs_ref, wn_ref, s_ref, z_ref):
    j = pl.program_id(1)

    @pl.when(j == 0)
    def _():
        s_ref[...] = jnp.maximum(x_ref[...] @ ws_ref[...] + bs_ref[...], 0.0)

    z_ref[...] = x_ref[...] @ wn_ref[...]


def _l2_body(s1_ref, alo_ref, ahi_ref, dega_ref, degb_ref, b1n_ref, ws_ref,
             bs_ref, wn_ref, s2_ref, z2_ref, n1_ref):
    j = pl.program_id(1)

    @pl.when(j == 0)
    def _():
        d = jnp.maximum(dega_ref[:, 0:1] + degb_ref[:, 0:1], 1.0)
        a = jnp.concatenate([alo_ref[...], ahi_ref[...]], axis=1) / d
        n1 = jnp.maximum(a + b1n_ref[...], 0.0)
        n1_ref[...] = n1
        ws = ws_ref[...]
        s2_ref[...] = jnp.maximum(
            s1_ref[...] @ ws[:256] + n1 @ ws[256:] + bs_ref[...], 0.0)

    wn = wn_ref[...]
    z2_ref[...] = s1_ref[...] @ wn[:256] + n1_ref[...] @ wn[256:]


def _l3_body(s2_ref, alo_ref, ahi_ref, dega_ref, degb_ref, b2n_ref, wc_ref,
             bc_ref, o_ref):
    d = jnp.maximum(dega_ref[:, 0:1] + degb_ref[:, 0:1], 1.0)
    a = jnp.concatenate([alo_ref[...], ahi_ref[...]], axis=1) / d
    n2 = jnp.maximum(a + b2n_ref[...], 0.0)
    s2 = s2_ref[...]
    ss = (jnp.sum(s2 * s2, axis=1, keepdims=True)
          + jnp.sum(n2 * n2, axis=1, keepdims=True))
    r = 1.0 / jnp.maximum(jnp.sqrt(ss), 1e-12)
    wc = wc_ref[...]
    o_ref[...] = (s2 * r) @ wc[:256] + (n2 * r) @ wc[256:] + bc_ref[...]


def kernel(x, edge_index, W1_self, b1_self, W1_neigh, b1_neigh,
           W2_self, b2_self, W2_neigh, b2_neigh, W_cls, b_cls):
    N, Din = x.shape
    E = edge_index.shape[1]
    H = W1_self.shape[1]
    Dout = W_cls.shape[1]
    F = H // 2
    NB = 10
    BM = N // NB
    G = (E // _NS) // _K
    GD = (E // (_NC * _NS)) // _KD

    src = edge_index[0]
    dst = edge_index[1]
    srccat = jnp.concatenate([src, src + N]).reshape(_NC, _NS, G, _K)
    dstr = dst.reshape(_NS, G, _K)
    dstr32 = dst.reshape(_NC, _NS, GD, _KD)

    full = lambda shape: pl.BlockSpec(shape, lambda i, j: (0, 0))

    deg2 = _make_sc_deg(N, E, F)(dstr32)

    s1, z1 = pl.pallas_call(
        _l1_body,
        grid=(NB, _NC),
        in_specs=[
            pl.BlockSpec((BM, Din), lambda i, j: (i, 0)),
            full((Din, H)),
            full((1, H)),
            pl.BlockSpec((Din, F), lambda i, j: (0, j)),
        ],
        out_specs=[
            pl.BlockSpec((BM, H), lambda i, j: (i, 0)),
            pl.BlockSpec((BM, F), lambda i, j: (j * NB + i, 0)),
        ],
        out_shape=[
            jax.ShapeDtypeStruct((N, H), jnp.float32),
            jax.ShapeDtypeStruct((2 * N, F), jnp.float32),
        ],
    )(x, W1_self, b1_self.reshape(1, H), W1_neigh)

    agg1 = _make_sc_agg(N, E, F)(z1, srccat, dstr)

    s2, z2 = pl.pallas_call(
        _l2_body,
        grid=(NB, _NC),
        in_specs=[
            pl.BlockSpec((BM, H), lambda i, j: (i, 0)),
            pl.BlockSpec((BM, F), lambda i, j: (i, 0)),
            pl.BlockSpec((BM, F), lambda i, j: (NB + i, 0)),
            pl.BlockSpec((BM, F), lambda i, j: (i, 0)),
            pl.BlockSpec((BM, F), lambda i, j: (NB + i, 0)),
            full((1, H)),
            full((2 * H, H)),
            full((1, H)),
            pl.BlockSpec((2 * H, F), lambda i, j: (0, j)),
        ],
        out_specs=[
            pl.BlockSpec((BM, H), lambda i, j: (i, 0)),
            pl.BlockSpec((BM, F), lambda i, j: (j * NB + i, 0)),
        ],
        out_shape=[
            jax.ShapeDtypeStruct((N, H), jnp.float32),
            jax.ShapeDtypeStruct((2 * N, F), jnp.float32),
        ],
        scratch_shapes=[pltpu.VMEM((BM, H), jnp.float32)],
    )(s1, agg1, agg1, deg2, deg2, b1_neigh.reshape(1, H), W2_self,
      b2_self.reshape(1, H), W2_neigh)

    agg2 = _make_sc_agg(N, E, F)(z2, srccat, dstr)

    out = pl.pallas_call(
        _l3_body,
        grid=(NB,),
        in_specs=[
            pl.BlockSpec((BM, H), lambda i: (i, 0)),
            pl.BlockSpec((BM, F), lambda i: (i, 0)),
            pl.BlockSpec((BM, F), lambda i: (NB + i, 0)),
            pl.BlockSpec((BM, F), lambda i: (i, 0)),
            pl.BlockSpec((BM, F), lambda i: (NB + i, 0)),
            pl.BlockSpec((1, H), lambda i: (0, 0)),
            pl.BlockSpec((2 * H, Dout), lambda i: (0, 0)),
            pl.BlockSpec((1, Dout), lambda i: (0, 0)),
        ],
        out_specs=pl.BlockSpec((BM, Dout), lambda i: (i, 0)),
        out_shape=jax.ShapeDtypeStruct((N, Dout), jnp.float32),
    )(s2, agg2, agg2, deg2, deg2, b2_neigh.reshape(1, H), W_cls,
      b_cls.reshape(1, Dout))

    return out


# speedup vs baseline: 2.3411x; 1.1811x over previous
import functools

import jax
import jax.numpy as jnp
from jax import lax
from jax.experimental import pallas as pl
from jax.experimental.pallas import tpu as pltpu
from jax.experimental.pallas import tpu_sc as plsc

_NC = 2
_NS = 16
_K = 80
_GP = 64
_KD = 40
_ZR = 24
_ZC = 48


def _zero_stripe(ref, zbuf, base, RP, N, t):
    @pl.loop(0, RP // _ZR)
    def _(r):
        pltpu.sync_copy(zbuf, ref.at[pl.ds(base + r * _ZR, _ZR)])

    tail = N - _NS * RP
    if tail:
        @pl.when(t == 0)
        def _():
            pltpu.sync_copy(zbuf.at[pl.ds(0, tail)],
                            ref.at[pl.ds(_NS * RP, tail)])


def _sc_agg_body(G, RP, N, F, z_hbm, srccat, dstr, out_hbm,
                 acc, srcb, dstb, rows0, rows1, sem0, sem1):
    c = lax.axis_index("c")
    t = lax.axis_index("s")
    base = t * RP
    tail = N - _NS * RP

    zero16 = jnp.zeros((16,), jnp.float32)

    @pl.loop(0, _ZC)
    def _(r):
        for c16 in range(F // 16):
            rows0[r, pl.ds(c16 * 16, 16)] = zero16

    @pl.loop(0, RP // _ZC)
    def _(r):
        pltpu.sync_copy(rows0.at[pl.ds(0, _ZC)],
                        acc.at[pl.ds(base + r * _ZC, _ZC)])

    if tail:
        @pl.when(t == 0)
        def _():
            pltpu.sync_copy(rows0.at[pl.ds(0, tail)],
                            acc.at[pl.ds(_NS * RP, tail)])

    plsc.subcore_barrier()

    for p in range((G + _GP - 1) // _GP):
        n = min(_GP, G - p * _GP)
        pltpu.sync_copy(srccat.at[c, t, pl.ds(p * _GP, n)],
                        srcb.at[pl.ds(0, n)])
        pltpu.sync_copy(dstr.at[t, pl.ds(p * _GP, n)],
                        dstb.at[pl.ds(0, n)])

        @pl.loop(0, n - (n % 2), step=2)
        def _(g):
            a = pltpu.async_copy(z_hbm.at[srcb.at[g]], rows0, sem0)
            b = pltpu.async_copy(z_hbm.at[srcb.at[g + 1]], rows1, sem1)
            a.wait()
            pltpu.sync_copy(rows0, acc.at[dstb.at[g]], add=True)
            b.wait()
            pltpu.sync_copy(rows1, acc.at[dstb.at[g + 1]], add=True)

        if n % 2:
            pltpu.async_copy(z_hbm.at[srcb.at[n - 1]], rows0, sem0).wait()
            pltpu.sync_copy(rows0, acc.at[dstb.at[n - 1]], add=True)

    plsc.subcore_barrier()

    pltpu.sync_copy(acc.at[pl.ds(base, RP)],
                    out_hbm.at[pl.ds(c * N + base, RP)])
    if tail:
        @pl.when(t == 0)
        def _():
            pltpu.sync_copy(acc.at[pl.ds(_NS * RP, tail)],
                            out_hbm.at[pl.ds(c * N + _NS * RP, tail)])


@functools.lru_cache(maxsize=None)
def _make_sc_agg(N, E, F):
    EP = E // _NS
    G = EP // _K
    RP = (N // _NS) // _ZC * _ZC

    mesh = plsc.VectorSubcoreMesh(core_axis_name="c", subcore_axis_name="s",
                                  num_cores=_NC, num_subcores=_NS)
    scratch = (
        pltpu.VMEM_SHARED((N, F), jnp.float32),
        pltpu.VMEM((_GP, _K), jnp.int32),
        pltpu.VMEM((_GP, _K), jnp.int32),
        pltpu.VMEM((_K, F), jnp.float32),
        pltpu.VMEM((_K, F), jnp.float32),
        pltpu.SemaphoreType.DMA,
        pltpu.SemaphoreType.DMA,
    )
    body = functools.partial(_sc_agg_body, G, RP, N, F)
    return pl.kernel(body,
                     out_type=jax.ShapeDtypeStruct((2 * N, F), jnp.float32),
                     mesh=mesh, scratch_types=scratch)


def _sc_deg_body(G, RP, N, F, dstr, deg_hbm, degsh, dstb, onesb, zbuf):
    c = lax.axis_index("c")
    t = lax.axis_index("s")
    base = t * RP
    tail = N - _NS * RP

    zero16 = jnp.zeros((16,), jnp.float32)
    one16 = jnp.full((16,), 1.0, jnp.float32)

    @pl.loop(0, _ZR)
    def _(r):
        for c16 in range(F // 16):
            zbuf[r, pl.ds(c16 * 16, 16)] = zero16

    @pl.loop(0, _KD)
    def _(r):
        for c16 in range(F // 16):
            onesb[r, pl.ds(c16 * 16, 16)] = one16

    _zero_stripe(degsh, zbuf, base, RP, N, t)

    plsc.subcore_barrier()

    pltpu.sync_copy(dstr.at[c, t], dstb)

    @pl.loop(0, G)
    def _(g):
        pltpu.sync_copy(onesb, degsh.at[dstb.at[g]], add=True)

    plsc.subcore_barrier()

    pltpu.sync_copy(degsh.at[pl.ds(base, RP)],
                    deg_hbm.at[pl.ds(c * N + base, RP)])
    if tail:
        @pl.when(t == 0)
        def _():
            pltpu.sync_copy(degsh.at[pl.ds(_NS * RP, tail)],
                            deg_hbm.at[pl.ds(c * N + _NS * RP, tail)])


@functools.lru_cache(maxsize=None)
def _make_sc_deg(N, E, F):
    EP = E // (_NC * _NS)
    G = EP // _KD
    RP = (N // _NS) // _ZR * _ZR

    mesh = plsc.VectorSubcoreMesh(core_axis_name="c", subcore_axis_name="s",
                                  num_cores=_NC, num_subcores=_NS)
    scratch = (
        pltpu.VMEM_SHARED((N, F), jnp.float32),
        pltpu.VMEM((G, _KD), jnp.int32),
        pltpu.VMEM((_KD, F), jnp.float32),
        pltpu.VMEM((_ZR, F), jnp.float32),
    )
    body = functools.partial(_sc_deg_body, G, RP, N, F)
    return pl.kernel(
        body,
        out_type=jax.ShapeDtypeStruct((_NC * N, F), jnp.float32),
        mesh=mesh, scratch_types=scratch)


def _l1_body(x_ref, ws_ref, bs_ref, wn_ref, s_ref, z_ref):
    j = pl.program_id(1)

    @pl.when(j == 0)
    def _():
        s_ref[...] = jnp.maximum(x_ref[...] @ ws_ref[...] + bs_ref[...], 0.0)

    z_ref[...] = x_ref[...] @ wn_ref[...]


def _l2_body(s1_ref, alo_ref, ahi_ref, dega_ref, degb_ref, b1n_ref, ws_ref,
             bs_ref, wn_ref, s2_ref, z2_ref, n1_ref):
    j = pl.program_id(1)

    @pl.when(j == 0)
    def _():
        d = jnp.maximum(dega_ref[:, 0:1] + degb_ref[:, 0:1], 1.0)
        a = jnp.concatenate([alo_ref[...], ahi_ref[...]], axis=1) / d
        n1 = jnp.maximum(a + b1n_ref[...], 0.0)
        n1_ref[...] = n1
        ws = ws_ref[...]
        s2_ref[...] = jnp.maximum(
            s1_ref[...] @ ws[:256] + n1 @ ws[256:] + bs_ref[...], 0.0)

    wn = wn_ref[...]
    z2_ref[...] = s1_ref[...] @ wn[:256] + n1_ref[...] @ wn[256:]


def _l3_body(s2_ref, alo_ref, ahi_ref, dega_ref, degb_ref, b2n_ref, wc_ref,
             bc_ref, o_ref):
    d = jnp.maximum(dega_ref[:, 0:1] + degb_ref[:, 0:1], 1.0)
    a = jnp.concatenate([alo_ref[...], ahi_ref[...]], axis=1) / d
    n2 = jnp.maximum(a + b2n_ref[...], 0.0)
    s2 = s2_ref[...]
    ss = (jnp.sum(s2 * s2, axis=1, keepdims=True)
          + jnp.sum(n2 * n2, axis=1, keepdims=True))
    r = 1.0 / jnp.maximum(jnp.sqrt(ss), 1e-12)
    wc = wc_ref[...]
    o_ref[...] = (s2 * r) @ wc[:256] + (n2 * r) @ wc[256:] + bc_ref[...]


def kernel(x, edge_index, W1_self, b1_self, W1_neigh, b1_neigh,
           W2_self, b2_self, W2_neigh, b2_neigh, W_cls, b_cls):
    N, Din = x.shape
    E = edge_index.shape[1]
    H = W1_self.shape[1]
    Dout = W_cls.shape[1]
    F = H // 2
    NB = 10
    BM = N // NB
    G = (E // _NS) // _K
    GD = (E // (_NC * _NS)) // _KD

    src = edge_index[0]
    dst = edge_index[1]
    srccat = jnp.concatenate([src, src + N]).reshape(_NC, _NS, G, _K)
    dstr = dst.reshape(_NS, G, _K)
    dstr32 = dst.reshape(_NC, _NS, GD, _KD)

    full = lambda shape: pl.BlockSpec(shape, lambda i, j: (0, 0))

    deg2 = _make_sc_deg(N, E, F)(dstr32)

    s1, z1 = pl.pallas_call(
        _l1_body,
        grid=(NB, _NC),
        in_specs=[
            pl.BlockSpec((BM, Din), lambda i, j: (i, 0)),
            full((Din, H)),
            full((1, H)),
            pl.BlockSpec((Din, F), lambda i, j: (0, j)),
        ],
        out_specs=[
            pl.BlockSpec((BM, H), lambda i, j: (i, 0)),
            pl.BlockSpec((BM, F), lambda i, j: (j * NB + i, 0)),
        ],
        out_shape=[
            jax.ShapeDtypeStruct((N, H), jnp.float32),
            jax.ShapeDtypeStruct((2 * N, F), jnp.float32),
        ],
    )(x, W1_self, b1_self.reshape(1, H), W1_neigh)

    agg1 = _make_sc_agg(N, E, F)(z1, srccat, dstr)

    s2, z2 = pl.pallas_call(
        _l2_body,
        grid=(NB, _NC),
        in_specs=[
            pl.BlockSpec((BM, H), lambda i, j: (i, 0)),
            pl.BlockSpec((BM, F), lambda i, j: (i, 0)),
            pl.BlockSpec((BM, F), lambda i, j: (NB + i, 0)),
            pl.BlockSpec((BM, F), lambda i, j: (i, 0)),
            pl.BlockSpec((BM, F), lambda i, j: (NB + i, 0)),
            full((1, H)),
            full((2 * H, H)),
            full((1, H)),
            pl.BlockSpec((2 * H, F), lambda i, j: (0, j)),
        ],
        out_specs=[
            pl.BlockSpec((BM, H), lambda i, j: (i, 0)),
            pl.BlockSpec((BM, F), lambda i, j: (j * NB + i, 0)),
        ],
        out_shape=[
            jax.ShapeDtypeStruct((N, H), jnp.float32),
            jax.ShapeDtypeStruct((2 * N, F), jnp.float32),
        ],
        scratch_shapes=[pltpu.VMEM((BM, H), jnp.float32)],
    )(s1, agg1, agg1, deg2, deg2, b1_neigh.reshape(1, H), W2_self,
      b2_self.reshape(1, H), W2_neigh)

    agg2 = _make_sc_agg(N, E, F)(z2, srccat, dstr)

    out = pl.pallas_call(
        _l3_body,
        grid=(NB,),
        in_specs=[
            pl.BlockSpec((BM, H), lambda i: (i, 0)),
            pl.BlockSpec((BM, F), lambda i: (i, 0)),
            pl.BlockSpec((BM, F), lambda i: (NB + i, 0)),
            pl.BlockSpec((BM, F), lambda i: (i, 0)),
            pl.BlockSpec((BM, F), lambda i: (NB + i, 0)),
            pl.BlockSpec((1, H), lambda i: (0, 0)),
            pl.BlockSpec((2 * H, Dout), lambda i: (0, 0)),
            pl.BlockSpec((1, Dout), lambda i: (0, 0)),
        ],
        out_specs=pl.BlockSpec((BM, Dout), lambda i: (i, 0)),
        out_shape=jax.ShapeDtypeStruct((N, Dout), jnp.float32),
    )(s2, agg2, agg2, deg2, deg2, b2_neigh.reshape(1, H), W_cls,
      b_cls.reshape(1, Dout))

    return out


# speedup vs baseline: 2.4219x; 1.0345x over previous
import functools

import jax
import jax.numpy as jnp
from jax import lax
from jax.experimental import pallas as pl
from jax.experimental.pallas import tpu as pltpu
from jax.experimental.pallas import tpu_sc as plsc

_NC = 2
_NS = 16
_K = 80
_GP = 64
_KD = 40
_ZR = 24
_ZC = 48


def _zero_stripe(ref, zbuf, base, RP, N, t):
    @pl.loop(0, RP // _ZR)
    def _(r):
        pltpu.sync_copy(zbuf, ref.at[pl.ds(base + r * _ZR, _ZR)])

    tail = N - _NS * RP
    if tail:
        @pl.when(t == 0)
        def _():
            pltpu.sync_copy(zbuf.at[pl.ds(0, tail)],
                            ref.at[pl.ds(_NS * RP, tail)])


def _sc_agg_body(G, RP, N, F, z_hbm, srccat, dstr, out_hbm,
                 acc, srcb, dstb, rows0, rows1, rows2, sem0, sem1, sem2):
    c = lax.axis_index("c")
    t = lax.axis_index("s")
    base = t * RP
    tail = N - _NS * RP

    zero16 = jnp.zeros((16,), jnp.float32)

    @pl.loop(0, _ZC)
    def _(r):
        for c16 in range(F // 16):
            rows0[r, pl.ds(c16 * 16, 16)] = zero16

    @pl.loop(0, RP // _ZC)
    def _(r):
        pltpu.sync_copy(rows0.at[pl.ds(0, _ZC)],
                        acc.at[pl.ds(base + r * _ZC, _ZC)])

    if tail:
        @pl.when(t == 0)
        def _():
            pltpu.sync_copy(rows0.at[pl.ds(0, tail)],
                            acc.at[pl.ds(_NS * RP, tail)])

    plsc.subcore_barrier()

    for p in range((G + _GP - 1) // _GP):
        n = min(_GP, G - p * _GP)
        pltpu.sync_copy(srccat.at[c, t, pl.ds(p * _GP, n)],
                        srcb.at[pl.ds(0, n)])
        pltpu.sync_copy(dstr.at[t, pl.ds(p * _GP, n)],
                        dstb.at[pl.ds(0, n)])

        @pl.loop(0, n - (n % 3), step=3)
        def _(g):
            a = pltpu.async_copy(z_hbm.at[srcb.at[g]], rows0, sem0)
            b = pltpu.async_copy(z_hbm.at[srcb.at[g + 1]], rows1, sem1)
            d = pltpu.async_copy(z_hbm.at[srcb.at[g + 2]], rows2, sem2)
            a.wait()
            pltpu.sync_copy(rows0, acc.at[dstb.at[g]], add=True)
            b.wait()
            pltpu.sync_copy(rows1, acc.at[dstb.at[g + 1]], add=True)
            d.wait()
            pltpu.sync_copy(rows2, acc.at[dstb.at[g + 2]], add=True)

        for q in range(n - (n % 3), n):
            pltpu.async_copy(z_hbm.at[srcb.at[q]], rows0, sem0).wait()
            pltpu.sync_copy(rows0, acc.at[dstb.at[q]], add=True)

    plsc.subcore_barrier()

    pltpu.sync_copy(acc.at[pl.ds(base, RP)],
                    out_hbm.at[pl.ds(c * N + base, RP)])
    if tail:
        @pl.when(t == 0)
        def _():
            pltpu.sync_copy(acc.at[pl.ds(_NS * RP, tail)],
                            out_hbm.at[pl.ds(c * N + _NS * RP, tail)])


@functools.lru_cache(maxsize=None)
def _make_sc_agg(N, E, F):
    EP = E // _NS
    G = EP // _K
    RP = (N // _NS) // _ZC * _ZC

    mesh = plsc.VectorSubcoreMesh(core_axis_name="c", subcore_axis_name="s",
                                  num_cores=_NC, num_subcores=_NS)
    scratch = (
        pltpu.VMEM_SHARED((N, F), jnp.float32),
        pltpu.VMEM((_GP, _K), jnp.int32),
        pltpu.VMEM((_GP, _K), jnp.int32),
        pltpu.VMEM((_K, F), jnp.float32),
        pltpu.VMEM((_K, F), jnp.float32),
        pltpu.VMEM((_K, F), jnp.float32),
        pltpu.SemaphoreType.DMA,
        pltpu.SemaphoreType.DMA,
        pltpu.SemaphoreType.DMA,
    )
    body = functools.partial(_sc_agg_body, G, RP, N, F)
    return pl.kernel(body,
                     out_type=jax.ShapeDtypeStruct((2 * N, F), jnp.float32),
                     mesh=mesh, scratch_types=scratch)


def _sc_deg_body(G, RP, N, F, dstr, deg_hbm, degsh, dstb, onesb, zbuf):
    c = lax.axis_index("c")
    t = lax.axis_index("s")
    base = t * RP
    tail = N - _NS * RP

    zero16 = jnp.zeros((16,), jnp.float32)
    one16 = jnp.full((16,), 1.0, jnp.float32)

    @pl.loop(0, _ZR)
    def _(r):
        for c16 in range(F // 16):
            zbuf[r, pl.ds(c16 * 16, 16)] = zero16

    @pl.loop(0, _KD)
    def _(r):
        for c16 in range(F // 16):
            onesb[r, pl.ds(c16 * 16, 16)] = one16

    _zero_stripe(degsh, zbuf, base, RP, N, t)

    plsc.subcore_barrier()

    pltpu.sync_copy(dstr.at[c, t], dstb)

    @pl.loop(0, G)
    def _(g):
        pltpu.sync_copy(onesb, degsh.at[dstb.at[g]], add=True)

    plsc.subcore_barrier()

    pltpu.sync_copy(degsh.at[pl.ds(base, RP)],
                    deg_hbm.at[pl.ds(c * N + base, RP)])
    if tail:
        @pl.when(t == 0)
        def _():
            pltpu.sync_copy(degsh.at[pl.ds(_NS * RP, tail)],
                            deg_hbm.at[pl.ds(c * N + _NS * RP, tail)])


@functools.lru_cache(maxsize=None)
def _make_sc_deg(N, E, F):
    EP = E // (_NC * _NS)
    G = EP // _KD
    RP = (N // _NS) // _ZR * _ZR

    mesh = plsc.VectorSubcoreMesh(core_axis_name="c", subcore_axis_name="s",
                                  num_cores=_NC, num_subcores=_NS)
    scratch = (
        pltpu.VMEM_SHARED((N, F), jnp.float32),
        pltpu.VMEM((G, _KD), jnp.int32),
        pltpu.VMEM((_KD, F), jnp.float32),
        pltpu.VMEM((_ZR, F), jnp.float32),
    )
    body = functools.partial(_sc_deg_body, G, RP, N, F)
    return pl.kernel(
        body,
        out_type=jax.ShapeDtypeStruct((_NC * N, F), jnp.float32),
        mesh=mesh, scratch_types=scratch)


def _l1_body(x_ref, ws_ref, bs_ref, wn_ref, s_ref, z_ref):
    j = pl.program_id(1)

    @pl.when(j == 0)
    def _():
        s_ref[...] = jnp.maximum(x_ref[...] @ ws_ref[...] + bs_ref[...], 0.0)

    z_ref[...] = x_ref[...] @ wn_ref[...]


def _l2_body(s1_ref, alo_ref, ahi_ref, dega_ref, degb_ref, b1n_ref, ws_ref,
             bs_ref, wn_ref, s2_ref, z2_ref, n1_ref):
    j = pl.program_id(1)

    @pl.when(j == 0)
    def _():
        d = jnp.maximum(dega_ref[:, 0:1] + degb_ref[:, 0:1], 1.0)
        a = jnp.concatenate([alo_ref[...], ahi_ref[...]], axis=1) / d
        n1 = jnp.maximum(a + b1n_ref[...], 0.0)
        n1_ref[...] = n1
        ws = ws_ref[...]
        s2_ref[...] = jnp.maximum(
            s1_ref[...] @ ws[:256] + n1 @ ws[256:] + bs_ref[...], 0.0)

    wn = wn_ref[...]
    z2_ref[...] = s1_ref[...] @ wn[:256] + n1_ref[...] @ wn[256:]


def _l3_body(s2_ref, alo_ref, ahi_ref, dega_ref, degb_ref, b2n_ref, wc_ref,
             bc_ref, o_ref):
    d = jnp.maximum(dega_ref[:, 0:1] + degb_ref[:, 0:1], 1.0)
    a = jnp.concatenate([alo_ref[...], ahi_ref[...]], axis=1) / d
    n2 = jnp.maximum(a + b2n_ref[...], 0.0)
    s2 = s2_ref[...]
    ss = (jnp.sum(s2 * s2, axis=1, keepdims=True)
          + jnp.sum(n2 * n2, axis=1, keepdims=True))
    r = 1.0 / jnp.maximum(jnp.sqrt(ss), 1e-12)
    wc = wc_ref[...]
    o_ref[...] = (s2 * r) @ wc[:256] + (n2 * r) @ wc[256:] + bc_ref[...]


def kernel(x, edge_index, W1_self, b1_self, W1_neigh, b1_neigh,
           W2_self, b2_self, W2_neigh, b2_neigh, W_cls, b_cls):
    N, Din = x.shape
    E = edge_index.shape[1]
    H = W1_self.shape[1]
    Dout = W_cls.shape[1]
    F = H // 2
    NB = 10
    BM = N // NB
    G = (E // _NS) // _K
    GD = (E // (_NC * _NS)) // _KD

    src = edge_index[0]
    dst = edge_index[1]
    srccat = jnp.concatenate([src, src + N]).reshape(_NC, _NS, G, _K)
    dstr = dst.reshape(_NS, G, _K)
    dstr32 = dst.reshape(_NC, _NS, GD, _KD)

    full = lambda shape: pl.BlockSpec(shape, lambda i, j: (0, 0))

    deg2 = _make_sc_deg(N, E, F)(dstr32)

    s1, z1 = pl.pallas_call(
        _l1_body,
        grid=(NB, _NC),
        in_specs=[
            pl.BlockSpec((BM, Din), lambda i, j: (i, 0)),
            full((Din, H)),
            full((1, H)),
            pl.BlockSpec((Din, F), lambda i, j: (0, j)),
        ],
        out_specs=[
            pl.BlockSpec((BM, H), lambda i, j: (i, 0)),
            pl.BlockSpec((BM, F), lambda i, j: (j * NB + i, 0)),
        ],
        out_shape=[
            jax.ShapeDtypeStruct((N, H), jnp.float32),
            jax.ShapeDtypeStruct((2 * N, F), jnp.float32),
        ],
    )(x, W1_self, b1_self.reshape(1, H), W1_neigh)

    agg1 = _make_sc_agg(N, E, F)(z1, srccat, dstr)

    s2, z2 = pl.pallas_call(
        _l2_body,
        grid=(NB, _NC),
        in_specs=[
            pl.BlockSpec((BM, H), lambda i, j: (i, 0)),
            pl.BlockSpec((BM, F), lambda i, j: (i, 0)),
            pl.BlockSpec((BM, F), lambda i, j: (NB + i, 0)),
            pl.BlockSpec((BM, F), lambda i, j: (i, 0)),
            pl.BlockSpec((BM, F), lambda i, j: (NB + i, 0)),
            full((1, H)),
            full((2 * H, H)),
            full((1, H)),
            pl.BlockSpec((2 * H, F), lambda i, j: (0, j)),
        ],
        out_specs=[
            pl.BlockSpec((BM, H), lambda i, j: (i, 0)),
            pl.BlockSpec((BM, F), lambda i, j: (j * NB + i, 0)),
        ],
        out_shape=[
            jax.ShapeDtypeStruct((N, H), jnp.float32),
            jax.ShapeDtypeStruct((2 * N, F), jnp.float32),
        ],
        scratch_shapes=[pltpu.VMEM((BM, H), jnp.float32)],
    )(s1, agg1, agg1, deg2, deg2, b1_neigh.reshape(1, H), W2_self,
      b2_self.reshape(1, H), W2_neigh)

    agg2 = _make_sc_agg(N, E, F)(z2, srccat, dstr)

    out = pl.pallas_call(
        _l3_body,
        grid=(NB,),
        in_specs=[
            pl.BlockSpec((BM, H), lambda i: (i, 0)),
            pl.BlockSpec((BM, F), lambda i: (i, 0)),
            pl.BlockSpec((BM, F), lambda i: (NB + i, 0)),
            pl.BlockSpec((BM, F), lambda i: (i, 0)),
            pl.BlockSpec((BM, F), lambda i: (NB + i, 0)),
            pl.BlockSpec((1, H), lambda i: (0, 0)),
            pl.BlockSpec((2 * H, Dout), lambda i: (0, 0)),
            pl.BlockSpec((1, Dout), lambda i: (0, 0)),
        ],
        out_specs=pl.BlockSpec((BM, Dout), lambda i: (i, 0)),
        out_shape=jax.ShapeDtypeStruct((N, Dout), jnp.float32),
    )(s2, agg2, agg2, deg2, deg2, b2_neigh.reshape(1, H), W_cls,
      b_cls.reshape(1, Dout))

    return out


# speedup vs baseline: 2.4405x; 1.0077x over previous
import functools

import jax
import jax.numpy as jnp
from jax import lax
from jax.experimental import pallas as pl
from jax.experimental.pallas import tpu as pltpu
from jax.experimental.pallas import tpu_sc as plsc

_NC = 2
_NS = 16
_K = 80
_GP = 64
_KD = 40
_ZR = 24
_ZC = 48


def _zero_stripe(ref, zbuf, base, RP, N, t):
    @pl.loop(0, RP // _ZR)
    def _(r):
        pltpu.sync_copy(zbuf, ref.at[pl.ds(base + r * _ZR, _ZR)])

    tail = N - _NS * RP
    if tail:
        @pl.when(t == 0)
        def _():
            pltpu.sync_copy(zbuf.at[pl.ds(0, tail)],
                            ref.at[pl.ds(_NS * RP, tail)])


def _sc_agg_body(G, RP, N, F, z_hbm, srccat, dstr, out_hbm,
                 acc, srcb, dstb, rows0, rows1, rows2, sem0, sem1, sem2):
    c = lax.axis_index("c")
    t = lax.axis_index("s")
    base = t * RP
    tail = N - _NS * RP

    zero16 = jnp.zeros((16,), jnp.float32)

    @pl.loop(0, _ZC)
    def _(r):
        for c16 in range(F // 16):
            rows0[r, pl.ds(c16 * 16, 16)] = zero16

    @pl.loop(0, RP // _ZC)
    def _(r):
        pltpu.sync_copy(rows0.at[pl.ds(0, _ZC)],
                        acc.at[pl.ds(base + r * _ZC, _ZC)])

    if tail:
        @pl.when(t == 0)
        def _():
            pltpu.sync_copy(rows0.at[pl.ds(0, tail)],
                            acc.at[pl.ds(_NS * RP, tail)])

    plsc.subcore_barrier()

    for p in range((G + _GP - 1) // _GP):
        n = min(_GP, G - p * _GP)
        pltpu.sync_copy(srccat.at[c, t, pl.ds(p * _GP, n)],
                        srcb.at[pl.ds(0, n)])
        pltpu.sync_copy(dstr.at[t, pl.ds(p * _GP, n)],
                        dstb.at[pl.ds(0, n)])

        @pl.loop(0, n - (n % 3), step=3)
        def _(g):
            a = pltpu.async_copy(z_hbm.at[srcb.at[g]], rows0, sem0)
            b = pltpu.async_copy(z_hbm.at[srcb.at[g + 1]], rows1, sem1)
            d = pltpu.async_copy(z_hbm.at[srcb.at[g + 2]], rows2, sem2)
            a.wait()
            pltpu.sync_copy(rows0, acc.at[dstb.at[g]], add=True)
            b.wait()
            pltpu.sync_copy(rows1, acc.at[dstb.at[g + 1]], add=True)
            d.wait()
            pltpu.sync_copy(rows2, acc.at[dstb.at[g + 2]], add=True)

        for q in range(n - (n % 3), n):
            pltpu.async_copy(z_hbm.at[srcb.at[q]], rows0, sem0).wait()
            pltpu.sync_copy(rows0, acc.at[dstb.at[q]], add=True)

    plsc.subcore_barrier()

    pltpu.sync_copy(acc.at[pl.ds(base, RP)],
                    out_hbm.at[pl.ds(c * N + base, RP)])
    if tail:
        @pl.when(t == 0)
        def _():
            pltpu.sync_copy(acc.at[pl.ds(_NS * RP, tail)],
                            out_hbm.at[pl.ds(c * N + _NS * RP, tail)])


@functools.lru_cache(maxsize=None)
def _make_sc_agg(N, E, F):
    EP = E // _NS
    G = EP // _K
    RP = (N // _NS) // _ZC * _ZC

    mesh = plsc.VectorSubcoreMesh(core_axis_name="c", subcore_axis_name="s",
                                  num_cores=_NC, num_subcores=_NS)
    scratch = (
        pltpu.VMEM_SHARED((N, F), jnp.float32),
        pltpu.VMEM((_GP, _K), jnp.int32),
        pltpu.VMEM((_GP, _K), jnp.int32),
        pltpu.VMEM((_K, F), jnp.float32),
        pltpu.VMEM((_K, F), jnp.float32),
        pltpu.VMEM((_K, F), jnp.float32),
        pltpu.SemaphoreType.DMA,
        pltpu.SemaphoreType.DMA,
        pltpu.SemaphoreType.DMA,
    )
    body = functools.partial(_sc_agg_body, G, RP, N, F)
    return pl.kernel(body,
                     out_type=jax.ShapeDtypeStruct((2 * N, F), jnp.float32),
                     mesh=mesh, scratch_types=scratch)


def _sc_deg_body(G, RP, N, F, dstr, deg_hbm, degsh, dstb, onesb, zbuf,
                 dsem0, dsem1):
    c = lax.axis_index("c")
    t = lax.axis_index("s")
    base = t * RP
    tail = N - _NS * RP

    zero16 = jnp.zeros((16,), jnp.float32)
    one16 = jnp.full((16,), 1.0, jnp.float32)

    @pl.loop(0, _ZR)
    def _(r):
        for c16 in range(F // 16):
            zbuf[r, pl.ds(c16 * 16, 16)] = zero16

    @pl.loop(0, _KD)
    def _(r):
        for c16 in range(F // 16):
            onesb[r, pl.ds(c16 * 16, 16)] = one16

    _zero_stripe(degsh, zbuf, base, RP, N, t)

    plsc.subcore_barrier()

    pltpu.sync_copy(dstr.at[c, t], dstb)

    @pl.loop(0, G - (G % 2), step=2)
    def _(g):
        a = pltpu.async_copy(onesb, degsh.at[dstb.at[g]], dsem0, add=True)
        b = pltpu.async_copy(onesb, degsh.at[dstb.at[g + 1]], dsem1,
                             add=True)
        a.wait()
        b.wait()

    for q in range(G - (G % 2), G):
        pltpu.sync_copy(onesb, degsh.at[dstb.at[q]], add=True)

    plsc.subcore_barrier()

    pltpu.sync_copy(degsh.at[pl.ds(base, RP)],
                    deg_hbm.at[pl.ds(c * N + base, RP)])
    if tail:
        @pl.when(t == 0)
        def _():
            pltpu.sync_copy(degsh.at[pl.ds(_NS * RP, tail)],
                            deg_hbm.at[pl.ds(c * N + _NS * RP, tail)])


@functools.lru_cache(maxsize=None)
def _make_sc_deg(N, E, F):
    EP = E // (_NC * _NS)
    G = EP // _KD
    RP = (N // _NS) // _ZR * _ZR

    mesh = plsc.VectorSubcoreMesh(core_axis_name="c", subcore_axis_name="s",
                                  num_cores=_NC, num_subcores=_NS)
    scratch = (
        pltpu.VMEM_SHARED((N, F), jnp.float32),
        pltpu.VMEM((G, _KD), jnp.int32),
        pltpu.VMEM((_KD, F), jnp.float32),
        pltpu.VMEM((_ZR, F), jnp.float32),
        pltpu.SemaphoreType.DMA,
        pltpu.SemaphoreType.DMA,
    )
    body = functools.partial(_sc_deg_body, G, RP, N, F)
    return pl.kernel(
        body,
        out_type=jax.ShapeDtypeStruct((_NC * N, F), jnp.float32),
        mesh=mesh, scratch_types=scratch)


def _l1_body(x_ref, ws_ref, bs_ref, wn_ref, s_ref, z_ref):
    j = pl.program_id(1)

    @pl.when(j == 0)
    def _():
        s_ref[...] = jnp.maximum(x_ref[...] @ ws_ref[...] + bs_ref[...], 0.0)

    z_ref[...] = x_ref[...] @ wn_ref[...]


def _l2_body(s1_ref, alo_ref, ahi_ref, dega_ref, degb_ref, b1n_ref, ws_ref,
             bs_ref, wn_ref, s2_ref, z2_ref, n1_ref):
    j = pl.program_id(1)

    @pl.when(j == 0)
    def _():
        d = jnp.maximum(dega_ref[:, 0:1] + degb_ref[:, 0:1], 1.0)
        a = jnp.concatenate([alo_ref[...], ahi_ref[...]], axis=1) / d
        n1 = jnp.maximum(a + b1n_ref[...], 0.0)
        n1_ref[...] = n1
        ws = ws_ref[...]
        s2_ref[...] = jnp.maximum(
            s1_ref[...] @ ws[:256] + n1 @ ws[256:] + bs_ref[...], 0.0)

    wn = wn_ref[...]
    z2_ref[...] = s1_ref[...] @ wn[:256] + n1_ref[...] @ wn[256:]


def _l3_body(s2_ref, alo_ref, ahi_ref, dega_ref, degb_ref, b2n_ref, wc_ref,
             bc_ref, o_ref):
    d = jnp.maximum(dega_ref[:, 0:1] + degb_ref[:, 0:1], 1.0)
    a = jnp.concatenate([alo_ref[...], ahi_ref[...]], axis=1) / d
    n2 = jnp.maximum(a + b2n_ref[...], 0.0)
    s2 = s2_ref[...]
    ss = (jnp.sum(s2 * s2, axis=1, keepdims=True)
          + jnp.sum(n2 * n2, axis=1, keepdims=True))
    r = 1.0 / jnp.maximum(jnp.sqrt(ss), 1e-12)
    wc = wc_ref[...]
    o_ref[...] = (s2 * r) @ wc[:256] + (n2 * r) @ wc[256:] + bc_ref[...]


def kernel(x, edge_index, W1_self, b1_self, W1_neigh, b1_neigh,
           W2_self, b2_self, W2_neigh, b2_neigh, W_cls, b_cls):
    N, Din = x.shape
    E = edge_index.shape[1]
    H = W1_self.shape[1]
    Dout = W_cls.shape[1]
    F = H // 2
    NB = 10
    BM = N // NB
    G = (E // _NS) // _K
    GD = (E // (_NC * _NS)) // _KD

    src = edge_index[0]
    dst = edge_index[1]
    srccat = jnp.concatenate([src, src + N]).reshape(_NC, _NS, G, _K)
    dstr = dst.reshape(_NS, G, _K)
    dstr32 = dst.reshape(_NC, _NS, GD, _KD)

    full = lambda shape: pl.BlockSpec(shape, lambda i, j: (0, 0))

    deg2 = _make_sc_deg(N, E, F)(dstr32)

    s1, z1 = pl.pallas_call(
        _l1_body,
        grid=(NB, _NC),
        in_specs=[
            pl.BlockSpec((BM, Din), lambda i, j: (i, 0)),
            full((Din, H)),
            full((1, H)),
            pl.BlockSpec((Din, F), lambda i, j: (0, j)),
        ],
        out_specs=[
            pl.BlockSpec((BM, H), lambda i, j: (i, 0)),
            pl.BlockSpec((BM, F), lambda i, j: (j * NB + i, 0)),
        ],
        out_shape=[
            jax.ShapeDtypeStruct((N, H), jnp.float32),
            jax.ShapeDtypeStruct((2 * N, F), jnp.float32),
        ],
    )(x, W1_self, b1_self.reshape(1, H), W1_neigh)

    agg1 = _make_sc_agg(N, E, F)(z1, srccat, dstr)

    s2, z2 = pl.pallas_call(
        _l2_body,
        grid=(NB, _NC),
        in_specs=[
            pl.BlockSpec((BM, H), lambda i, j: (i, 0)),
            pl.BlockSpec((BM, F), lambda i, j: (i, 0)),
            pl.BlockSpec((BM, F), lambda i, j: (NB + i, 0)),
            pl.BlockSpec((BM, F), lambda i, j: (i, 0)),
            pl.BlockSpec((BM, F), lambda i, j: (NB + i, 0)),
            full((1, H)),
            full((2 * H, H)),
            full((1, H)),
            pl.BlockSpec((2 * H, F), lambda i, j: (0, j)),
        ],
        out_specs=[
            pl.BlockSpec((BM, H), lambda i, j: (i, 0)),
            pl.BlockSpec((BM, F), lambda i, j: (j * NB + i, 0)),
        ],
        out_shape=[
            jax.ShapeDtypeStruct((N, H), jnp.float32),
            jax.ShapeDtypeStruct((2 * N, F), jnp.float32),
        ],
        scratch_shapes=[pltpu.VMEM((BM, H), jnp.float32)],
    )(s1, agg1, agg1, deg2, deg2, b1_neigh.reshape(1, H), W2_self,
      b2_self.reshape(1, H), W2_neigh)

    agg2 = _make_sc_agg(N, E, F)(z2, srccat, dstr)

    out = pl.pallas_call(
        _l3_body,
        grid=(NB,),
        in_specs=[
            pl.BlockSpec((BM, H), lambda i: (i, 0)),
            pl.BlockSpec((BM, F), lambda i: (i, 0)),
            pl.BlockSpec((BM, F), lambda i: (NB + i, 0)),
            pl.BlockSpec((BM, F), lambda i: (i, 0)),
            pl.BlockSpec((BM, F), lambda i: (NB + i, 0)),
            pl.BlockSpec((1, H), lambda i: (0, 0)),
            pl.BlockSpec((2 * H, Dout), lambda i: (0, 0)),
            pl.BlockSpec((1, Dout), lambda i: (0, 0)),
        ],
        out_specs=pl.BlockSpec((BM, Dout), lambda i: (i, 0)),
        out_shape=jax.ShapeDtypeStruct((N, Dout), jnp.float32),
    )(s2, agg2, agg2, deg2, deg2, b2_neigh.reshape(1, H), W_cls,
      b_cls.reshape(1, Dout))

    return out
